# Initial kernel scaffold; baseline (speedup 1.0000x reference)
import jax
import jax.numpy as jnp
from jax.experimental import pallas as pl


def kernel(inputs, targets):
    raise NotImplementedError("write your pallas kernel here")


# speedup vs baseline: 1.7521x; 1.7521x over previous
import jax
import jax.numpy as jnp
from jax.experimental import pallas as pl
from jax.experimental.pallas import tpu as pltpu

ALPHA = 16.0
N = 4096
R = 256
G = N // R


def _nca_body(x_row_ref, t_row_ref, x_col_ref, t_col_ref,
              loss_ref, prec_ref, mps_ref, mns_ref):
    i = pl.program_id(0)

    x_row = x_row_ref[...]
    t_row = t_row_ref[...]
    x_col = x_col_ref[...]
    t_col = t_col_ref[...]

    sim = jnp.abs(x_col - x_row)
    pos = t_col == t_row
    neg = jnp.logical_not(pos)
    pos_valid = pos & (sim < 1.0)

    sel = pos_valid | neg
    thr = jnp.max(jnp.where(sel, sim, -1.0), axis=1, keepdims=True)

    below = sim < thr
    pos_neig = pos_valid & below
    neg_neig = neg & below
    has_pos = jnp.any(pos_neig, axis=1, keepdims=True)

    base = jnp.mean(sim, axis=1, keepdims=True)
    w = jnp.exp(ALPHA * (base - sim))
    p_neig = jnp.sum(jnp.where(pos_neig, w, 0.0), axis=1, keepdims=True)
    p_valid = jnp.sum(jnp.where(pos_valid, w, 0.0), axis=1, keepdims=True)
    p = jnp.where(has_pos, p_neig, p_valid)
    q = jnp.sum(jnp.where(neg_neig, w, 0.0), axis=1, keepdims=True)

    loss_i = -jnp.log(p / (p + q))

    @pl.when(i == 0)
    def _init():
        loss_ref[...] = jnp.zeros_like(loss_ref)
        prec_ref[...] = jnp.zeros_like(prec_ref)

    loss_ref[...] += jnp.sum(loss_i).reshape(1, 1)
    prec_ref[...] += jnp.sum(jnp.where(loss_i < 0.6, 1.0, 0.0)).reshape(1, 1)

    @pl.when(i == G - 1)
    def _last():
        s = sim[R - 1:R, :]
        lp = jnp.where(pos_valid[R - 1:R, :], 1.0, 0.0)
        ln = jnp.where(neg[R - 1:R, :], 1.0, 0.0)
        mps_ref[...] = (jnp.sum(s * lp) / jnp.sum(lp)).reshape(1, 1)
        mns_ref[...] = (jnp.sum(s * ln) / jnp.sum(ln)).reshape(1, 1)
        loss_ref[...] = loss_ref[...] * (1.0 / N)
        prec_ref[...] = prec_ref[...] * (1.0 / N)


def kernel(inputs, targets):
    t32 = targets.astype(jnp.int32)
    x_rows = inputs.reshape(N, 1)
    t_rows = t32.reshape(N, 1)
    x_cols = inputs.reshape(1, N)
    t_cols = t32.reshape(1, N)

    out = pl.pallas_call(
        _nca_body,
        grid=(G,),
        in_specs=[
            pl.BlockSpec((R, 1), lambda i: (i, 0)),
            pl.BlockSpec((R, 1), lambda i: (i, 0)),
            pl.BlockSpec((1, N), lambda i: (0, 0)),
            pl.BlockSpec((1, N), lambda i: (0, 0)),
        ],
        out_specs=[
            pl.BlockSpec((1, 1), lambda i: (0, 0)),
            pl.BlockSpec((1, 1), lambda i: (0, 0)),
            pl.BlockSpec((1, 1), lambda i: (0, 0)),
            pl.BlockSpec((1, 1), lambda i: (0, 0)),
        ],
        out_shape=[jax.ShapeDtypeStruct((1, 1), jnp.float32)] * 4,
    )(x_rows, t_rows, x_cols, t_cols)

    loss, prec, mps, mns = out
    return (loss[0, 0], prec[0, 0], mps[0, 0], mns[0, 0])


# speedup vs baseline: 2.0845x; 1.1897x over previous
import jax
import jax.numpy as jnp
from jax.experimental import pallas as pl
from jax.experimental.pallas import tpu as pltpu

ALPHA = 16.0
N = 4096
R = 256
G = N // R


def _nca_body(x_row_ref, t_row_ref, x_col_ref, t_col_ref,
              loss_ref, prec_ref, mps_ref, mns_ref):
    i = pl.program_id(0)

    x_row = x_row_ref[...]
    t_row = t_row_ref[...]
    x_col = x_col_ref[...]
    t_col = t_col_ref[...]

    sim = jnp.abs(x_col - x_row)
    pos = t_col == t_row
    neg = jnp.logical_not(pos)
    pos_valid = pos & (sim < 1.0)

    sel = pos_valid | neg
    thr = jnp.max(jnp.where(sel, sim, -1.0), axis=1, keepdims=True)

    below = sim < thr
    pos_neig = pos_valid & below
    neg_neig = neg & below

    w = jnp.exp(-ALPHA * sim)
    p_neig = jnp.sum(jnp.where(pos_neig, w, 0.0), axis=1, keepdims=True)
    p_valid = jnp.sum(jnp.where(pos_valid, w, 0.0), axis=1, keepdims=True)
    p = jnp.where(p_neig > 0.0, p_neig, p_valid)
    q = jnp.sum(jnp.where(neg_neig, w, 0.0), axis=1, keepdims=True)

    loss_i = -jnp.log(p / (p + q))

    @pl.when(i == 0)
    def _init():
        loss_ref[...] = jnp.zeros_like(loss_ref)
        prec_ref[...] = jnp.zeros_like(prec_ref)

    loss_ref[...] += jnp.sum(loss_i).reshape(1, 1)
    prec_ref[...] += jnp.sum(jnp.where(loss_i < 0.6, 1.0, 0.0)).reshape(1, 1)

    @pl.when(i == G - 1)
    def _last():
        s = sim[R - 1:R, :]
        lp = jnp.where(pos_valid[R - 1:R, :], 1.0, 0.0)
        ln = jnp.where(neg[R - 1:R, :], 1.0, 0.0)
        mps_ref[...] = (jnp.sum(s * lp) / jnp.sum(lp)).reshape(1, 1)
        mns_ref[...] = (jnp.sum(s * ln) / jnp.sum(ln)).reshape(1, 1)
        loss_ref[...] = loss_ref[...] * (1.0 / N)
        prec_ref[...] = prec_ref[...] * (1.0 / N)


def kernel(inputs, targets):
    t32 = targets.astype(jnp.int32)
    x_rows = inputs.reshape(N, 1)
    t_rows = t32.reshape(N, 1)
    x_cols = inputs.reshape(1, N)
    t_cols = t32.reshape(1, N)

    out = pl.pallas_call(
        _nca_body,
        grid=(G,),
        in_specs=[
            pl.BlockSpec((R, 1), lambda i: (i, 0)),
            pl.BlockSpec((R, 1), lambda i: (i, 0)),
            pl.BlockSpec((1, N), lambda i: (0, 0)),
            pl.BlockSpec((1, N), lambda i: (0, 0)),
        ],
        out_specs=[
            pl.BlockSpec((1, 1), lambda i: (0, 0)),
            pl.BlockSpec((1, 1), lambda i: (0, 0)),
            pl.BlockSpec((1, 1), lambda i: (0, 0)),
            pl.BlockSpec((1, 1), lambda i: (0, 0)),
        ],
        out_shape=[jax.ShapeDtypeStruct((1, 1), jnp.float32)] * 4,
    )(x_rows, t_rows, x_cols, t_cols)

    loss, prec, mps, mns = out
    return (loss[0, 0], prec[0, 0], mps[0, 0], mns[0, 0])
